# baseline (device time: 571512 ns/iter reference)
import jax
import jax.numpy as jnp
from jax import lax
from jax.experimental import pallas as pl
from jax.experimental.pallas import tpu as pltpu

N_DEV = 32


def kernel(x, w_mat, scale_x, scale_w):
    m_glob, k_per = x.shape
    _, n = w_mat.shape
    m_chunk = m_glob // N_DEV

    def body(x_ref, w_ref, sx_ref, sw_ref, out_ref,
             comm_ref, send_sems, recv_sems, credit_sem):
        my = lax.axis_index("i")
        left = lax.rem(my + N_DEV - 1, N_DEV)
        right = lax.rem(my + 1, N_DEV)

        barrier = pltpu.get_barrier_semaphore()
        for nbr in (left, right):
            pl.semaphore_signal(
                barrier, inc=1,
                device_id=(nbr,), device_id_type=pl.DeviceIdType.MESH,
            )
        pl.semaphore_wait(barrier, 2)

        wb = w_ref[:, :].astype(jnp.bfloat16)

        def partial(c):
            xs = x_ref[pl.ds(c * m_chunk, m_chunk), :].astype(jnp.bfloat16)
            return jnp.dot(xs, wb, preferred_element_type=jnp.float32)

        comm_ref[0, :, :] = partial(lax.rem(my + N_DEV - 1, N_DEV))

        for h in range(N_DEV - 1):
            s_slot = h % 2
            r_slot = (h + 1) % 2
            if h >= 1:
                pl.semaphore_wait(credit_sem, 1)
            rdma = pltpu.make_async_remote_copy(
                src_ref=comm_ref.at[s_slot],
                dst_ref=comm_ref.at[r_slot],
                send_sem=send_sems.at[s_slot],
                recv_sem=recv_sems.at[r_slot],
                device_id=(right,),
                device_id_type=pl.DeviceIdType.MESH,
            )
            rdma.start()
            c_in = lax.rem(my + 2 * N_DEV - h - 2, N_DEV)
            p = partial(c_in)
            rdma.wait()
            if h < N_DEV - 2:
                pl.semaphore_signal(
                    credit_sem, inc=1,
                    device_id=(left,), device_id_type=pl.DeviceIdType.MESH,
                )
                comm_ref[r_slot, :, :] = comm_ref[r_slot, :, :] + p
            else:
                acc = comm_ref[r_slot, :, :] + p
                y = acc * (sx_ref[0] * sw_ref[0])
                out_ref[:, :] = y * (1.0 / (1.0 + jnp.exp(-y)))

    out_shape = jax.ShapeDtypeStruct((m_chunk, n), jnp.float32)
    return pl.pallas_call(
        body,
        out_shape=out_shape,
        in_specs=[
            pl.BlockSpec(memory_space=pltpu.VMEM),
            pl.BlockSpec(memory_space=pltpu.VMEM),
            pl.BlockSpec(memory_space=pltpu.SMEM),
            pl.BlockSpec(memory_space=pltpu.SMEM),
        ],
        out_specs=pl.BlockSpec(memory_space=pltpu.VMEM),
        scratch_shapes=[
            pltpu.VMEM((2, m_chunk, n), jnp.float32),
            pltpu.SemaphoreType.DMA((2,)),
            pltpu.SemaphoreType.DMA((2,)),
            pltpu.SemaphoreType.REGULAR,
        ],
        compiler_params=pltpu.CompilerParams(collective_id=0),
    )(x, w_mat, scale_x, scale_w)


# device time: 419066 ns/iter; 1.3638x vs baseline; 1.3638x over previous
import jax
import jax.numpy as jnp
from jax import lax
from jax.experimental import pallas as pl
from jax.experimental.pallas import tpu as pltpu

N_DEV = 32


def kernel(x, w_mat, scale_x, scale_w):
    m_glob, k_per = x.shape
    _, n = w_mat.shape
    m_chunk = m_glob // N_DEV
    nh = n // 2

    def body(x_ref, w_ref, sx_ref, sw_ref, out_ref,
             comm_r, comm_l, send_r, recv_r, send_l, recv_l,
             credit_r, credit_l):
        my = lax.axis_index("i")
        left = lax.rem(my + N_DEV - 1, N_DEV)
        right = lax.rem(my + 1, N_DEV)

        barrier = pltpu.get_barrier_semaphore()
        for nbr in (left, right):
            pl.semaphore_signal(
                barrier, inc=1,
                device_id=(nbr,), device_id_type=pl.DeviceIdType.MESH,
            )
        pl.semaphore_wait(barrier, 2)

        wbr = w_ref[:, :nh].astype(jnp.bfloat16)
        wbl = w_ref[:, nh:].astype(jnp.bfloat16)

        def xchunk(c):
            return x_ref[pl.ds(c * m_chunk, m_chunk), :].astype(jnp.bfloat16)

        comm_r[0, :, :] = jnp.dot(
            xchunk(lax.rem(my + N_DEV - 1, N_DEV)), wbr,
            preferred_element_type=jnp.float32)
        comm_l[0, :, :] = jnp.dot(
            xchunk(lax.rem(my + 1, N_DEV)), wbl,
            preferred_element_type=jnp.float32)

        for h in range(N_DEV - 1):
            s_slot = h % 2
            r_slot = (h + 1) % 2
            if h >= 1:
                pl.semaphore_wait(credit_r, 1)
                pl.semaphore_wait(credit_l, 1)
            rdma_r = pltpu.make_async_remote_copy(
                src_ref=comm_r.at[s_slot],
                dst_ref=comm_r.at[r_slot],
                send_sem=send_r.at[s_slot],
                recv_sem=recv_r.at[r_slot],
                device_id=(right,),
                device_id_type=pl.DeviceIdType.MESH,
            )
            rdma_l = pltpu.make_async_remote_copy(
                src_ref=comm_l.at[s_slot],
                dst_ref=comm_l.at[r_slot],
                send_sem=send_l.at[s_slot],
                recv_sem=recv_l.at[r_slot],
                device_id=(left,),
                device_id_type=pl.DeviceIdType.MESH,
            )
            rdma_r.start()
            rdma_l.start()
            c_r = lax.rem(my + 2 * N_DEV - h - 2, N_DEV)
            c_l = lax.rem(my + h + 2, N_DEV)
            p_r = jnp.dot(xchunk(c_r), wbr, preferred_element_type=jnp.float32)
            p_l = jnp.dot(xchunk(c_l), wbl, preferred_element_type=jnp.float32)
            rdma_r.wait()
            rdma_l.wait()
            if h < N_DEV - 2:
                pl.semaphore_signal(
                    credit_r, inc=1,
                    device_id=(left,), device_id_type=pl.DeviceIdType.MESH,
                )
                pl.semaphore_signal(
                    credit_l, inc=1,
                    device_id=(right,), device_id_type=pl.DeviceIdType.MESH,
                )
                comm_r[r_slot, :, :] = comm_r[r_slot, :, :] + p_r
                comm_l[r_slot, :, :] = comm_l[r_slot, :, :] + p_l
            else:
                scale = sx_ref[0] * sw_ref[0]
                acc_r = comm_r[r_slot, :, :] + p_r
                y = acc_r * scale
                out_ref[:, :nh] = y * (1.0 / (1.0 + jnp.exp(-y)))
                acc_l = comm_l[r_slot, :, :] + p_l
                y = acc_l * scale
                out_ref[:, nh:] = y * (1.0 / (1.0 + jnp.exp(-y)))

    out_shape = jax.ShapeDtypeStruct((m_chunk, n), jnp.float32)
    return pl.pallas_call(
        body,
        out_shape=out_shape,
        in_specs=[
            pl.BlockSpec(memory_space=pltpu.VMEM),
            pl.BlockSpec(memory_space=pltpu.VMEM),
            pl.BlockSpec(memory_space=pltpu.SMEM),
            pl.BlockSpec(memory_space=pltpu.SMEM),
        ],
        out_specs=pl.BlockSpec(memory_space=pltpu.VMEM),
        scratch_shapes=[
            pltpu.VMEM((2, m_chunk, nh), jnp.float32),
            pltpu.VMEM((2, m_chunk, nh), jnp.float32),
            pltpu.SemaphoreType.DMA((2,)),
            pltpu.SemaphoreType.DMA((2,)),
            pltpu.SemaphoreType.DMA((2,)),
            pltpu.SemaphoreType.DMA((2,)),
            pltpu.SemaphoreType.REGULAR,
            pltpu.SemaphoreType.REGULAR,
        ],
        compiler_params=pltpu.CompilerParams(collective_id=0),
    )(x, w_mat, scale_x, scale_w)
